# baseline (speedup 1.0000x reference)
import functools

import jax
import jax.numpy as jnp
from jax.experimental import pallas as pl
from jax.experimental.pallas import tpu as pltpu


def _pick_block(n: int, target: int) -> int:
    best = None
    for b in range(8, min(n, target) + 1, 8):
        if n % b == 0:
            best = b
    return best if best is not None else n


def _xw_body(x_ref, w_ref, h_ref):
    xb = x_ref[...].astype(jnp.bfloat16)
    wb = w_ref[...].astype(jnp.bfloat16)
    h_ref[...] = jnp.dot(
        xb, wb, preferred_element_type=jnp.float32
    ).astype(jnp.bfloat16)


def _agg_body(adj_ref, h_ref, b_ref, out_ref):
    acc = jnp.dot(
        adj_ref[...], h_ref[...], preferred_element_type=jnp.float32
    )
    out_ref[...] = acc + b_ref[...]


@jax.jit
def kernel(x, adj_mat, weight, bias):
    n, d_in = x.shape
    d_out = weight.shape[1]

    bm1 = _pick_block(n, 2000)
    h = pl.pallas_call(
        _xw_body,
        grid=(n // bm1,),
        in_specs=[
            pl.BlockSpec((bm1, d_in), lambda i: (i, 0)),
            pl.BlockSpec((d_in, d_out), lambda i: (0, 0)),
        ],
        out_specs=pl.BlockSpec((bm1, d_out), lambda i: (i, 0)),
        out_shape=jax.ShapeDtypeStruct((n, d_out), jnp.bfloat16),
        compiler_params=pltpu.CompilerParams(
            dimension_semantics=("parallel",),
        ),
    )(x, weight)

    bm2 = _pick_block(n, 200)
    bias2 = bias.reshape(1, d_out)
    out = pl.pallas_call(
        _agg_body,
        grid=(n // bm2,),
        in_specs=[
            pl.BlockSpec((bm2, n), lambda i: (i, 0)),
            pl.BlockSpec((n, d_out), lambda i: (0, 0)),
            pl.BlockSpec((1, d_out), lambda i: (0, 0)),
        ],
        out_specs=pl.BlockSpec((bm2, d_out), lambda i: (i, 0)),
        out_shape=jax.ShapeDtypeStruct((n, d_out), jnp.float32),
        compiler_params=pltpu.CompilerParams(
            dimension_semantics=("parallel",),
            vmem_limit_bytes=100 * 1024 * 1024,
        ),
    )(adj_mat, h, bias2)
    return out


# speedup vs baseline: 1.1548x; 1.1548x over previous
import jax
import jax.numpy as jnp
from jax.experimental import pallas as pl
from jax.experimental.pallas import tpu as pltpu


def _pick_block(n: int, target: int) -> int:
    best = None
    for b in range(8, min(n, target) + 1, 8):
        if n % b == 0:
            best = b
    return best if best is not None else n


def _gcn_body(adj_ref, x_ref, w_ref, b_ref, out_ref):
    g = jnp.dot(adj_ref[...], x_ref[...], preferred_element_type=jnp.float32)
    out_ref[...] = (
        jnp.dot(g, w_ref[...], preferred_element_type=jnp.float32)
        + b_ref[...]
    )


@jax.jit
def kernel(x, adj_mat, weight, bias):
    n, d_in = x.shape
    d_out = weight.shape[1]
    bm = _pick_block(n, 400)
    bias2 = bias.reshape(1, d_out)
    out = pl.pallas_call(
        _gcn_body,
        grid=(n // bm,),
        in_specs=[
            pl.BlockSpec((bm, n), lambda i: (i, 0)),
            pl.BlockSpec((n, d_in), lambda i: (0, 0)),
            pl.BlockSpec((d_in, d_out), lambda i: (0, 0)),
            pl.BlockSpec((1, d_out), lambda i: (0, 0)),
        ],
        out_specs=pl.BlockSpec((bm, d_out), lambda i: (i, 0)),
        out_shape=jax.ShapeDtypeStruct((n, d_out), jnp.float32),
        compiler_params=pltpu.CompilerParams(
            dimension_semantics=("arbitrary",),
            vmem_limit_bytes=128 * 1024 * 1024,
        ),
    )(adj_mat, x, weight, bias2)
    return out
